# baseline (speedup 1.0000x reference)
import functools

import jax
import jax.numpy as jnp
from jax import lax
from jax.experimental import pallas as pl
from jax.experimental.pallas import tpu as pltpu
from jax.experimental.pallas import tpu_sc as plsc

NC = 2
NS = 16
NW = NC * NS
L = 16


def _sc_scatter_gather(n_nodes, n_edges, d, c_sz, sc_chunks):
    epw = n_edges // NW
    n_chunks = epw // c_sz
    n_super = n_chunks // sc_chunks
    s_sz = sc_chunks * c_sz
    rows_pt = (n_nodes // NS) // 8 * 8
    tail = n_nodes - rows_pt * NS
    zrows = 16
    mesh = plsc.VectorSubcoreMesh(core_axis_name="c", subcore_axis_name="s")

    @functools.partial(
        pl.kernel,
        out_type=jax.ShapeDtypeStruct((NC * n_nodes, d), jnp.float32),
        mesh=mesh,
        scratch_types=[
            pltpu.VMEM((sc_chunks, c_sz), jnp.int32),
            pltpu.VMEM((sc_chunks, c_sz), jnp.int32),
            pltpu.VMEM((s_sz,), jnp.float32),
            pltpu.VMEM((s_sz,), jnp.float32),
            pltpu.VMEM((c_sz, d), jnp.float32),
            pltpu.VMEM((c_sz, d), jnp.float32),
            pltpu.VMEM((16, d), jnp.float32),
            pltpu.VMEM_SHARED((n_nodes, d), jnp.float32),
            pltpu.SemaphoreType.DMA,
            pltpu.SemaphoreType.DMA,
            pltpu.SemaphoreType.DMA,
            pltpu.SemaphoreType.DMA,
        ],
    )
    def k(inp_hbm, sidx_hbm, tidx_hbm, enorm_hbm, esgn_hbm, out_hbm,
          sidx_v, tidx_v, w_v, sg_v, rows_a, rows_b, zbuf, acc,
          gs_a, gs_b, ss_a, ss_b):
        cid = lax.axis_index("c")
        sid = lax.axis_index("s")
        wid = sid * NC + cid
        row0 = pl.multiple_of(sid * rows_pt, 8)
        scope = jax.named_scope

        sc_init = scope("sc_init")
        sc_init.__enter__()
        def zero_zbuf(r, _):
            for f in range(d // L):
                zbuf[r, pl.ds(f * L, L)] = jnp.zeros((L,), jnp.float32)
            return 0
        lax.fori_loop(0, zrows, zero_zbuf, 0)

        def zero_acc(i, _):
            pltpu.sync_copy(
                zbuf, acc.at[pl.ds(pl.multiple_of(row0 + i * zrows, 8), zrows)])
            return 0
        lax.fori_loop(0, rows_pt // zrows, zero_acc, 0)
        if tail:
            @pl.when(sid == NS - 1)
            def _():
                pltpu.sync_copy(zbuf.at[pl.ds(0, tail)],
                                acc.at[pl.ds(NS * rows_pt, tail)])

        plsc.subcore_barrier()
        sc_init.__exit__(None, None, None)

        def scale(rows_v, c):
            def escale(g, _):
                w16 = w_v[pl.ds(c * c_sz + g * L, L)]
                for j in range(L):
                    wb = lax.gather(
                        w16, jnp.full((L, 1), j, jnp.int32),
                        lax.GatherDimensionNumbers(
                            offset_dims=(), collapsed_slice_dims=(0,),
                            start_index_map=(0,)),
                        (1,), mode=lax.GatherScatterMode.PROMISE_IN_BOUNDS)
                    e = g * L + j
                    for f in range(d // L):
                        rows_v[e, pl.ds(f * L, L)] = (
                            rows_v[e, pl.ds(f * L, L)] * wb)
                return 0
            lax.fori_loop(0, c_sz // L, escale, 0)

        def issue_gather(c, rows_v, sem):
            pltpu.async_copy(inp_hbm.at[sidx_v.at[c]], rows_v, sem)

        def wait_gather(rows_v, sem):
            pltpu.make_async_copy(inp_hbm.at[sidx_v.at[0]], rows_v, sem).wait()

        def issue_scatter(c, rows_v, sem):
            pltpu.async_copy(rows_v, acc.at[tidx_v.at[c]], sem, add=True)

        def wait_scatter(rows_v, sem):
            pltpu.make_async_copy(rows_v, acc.at[tidx_v.at[0]], sem).wait()

        def super_round(s, _):
            with scope("sc_stage"):
                pltpu.sync_copy(sidx_hbm.at[wid, s], sidx_v)
                pltpu.sync_copy(tidx_hbm.at[wid, s], tidx_v)
                e0 = pl.multiple_of(wid * epw + s * s_sz, 8)
                pltpu.sync_copy(enorm_hbm.at[pl.ds(e0, s_sz)], w_v)
                pltpu.sync_copy(esgn_hbm.at[pl.ds(e0, s_sz)], sg_v)

                def wmul(kk, _):
                    w_v[pl.ds(kk * L, L)] = (
                        w_v[pl.ds(kk * L, L)] * sg_v[pl.ds(kk * L, L)])
                    return 0
                lax.fori_loop(0, s_sz // L, wmul, 0)

            issue_gather(0, rows_a, gs_a)

            sc_chunkloop = scope("sc_chunks")
            sc_chunkloop.__enter__()
            def pair(c2, _):
                ca = 2 * c2
                cb = 2 * c2 + 1

                wait_gather(rows_a, gs_a)
                scale(rows_a, ca)
                @pl.when(cb < sc_chunks)
                def _():
                    @pl.when(ca >= 2)
                    def _():
                        wait_scatter(rows_b, ss_b)
                    issue_gather(cb, rows_b, gs_b)
                issue_scatter(ca, rows_a, ss_a)

                @pl.when(cb < sc_chunks)
                def _():
                    wait_gather(rows_b, gs_b)
                    scale(rows_b, cb)
                    @pl.when(cb + 1 < sc_chunks)
                    def _():
                        wait_scatter(rows_a, ss_a)
                        issue_gather(cb + 1, rows_a, gs_a)
                    issue_scatter(cb, rows_b, ss_b)
                return 0
            lax.fori_loop(0, (sc_chunks + 1) // 2, pair, 0)

            wait_scatter(rows_a, ss_a)
            if sc_chunks > 1:
                wait_scatter(rows_b, ss_b)
            sc_chunkloop.__exit__(None, None, None)
            return 0
        lax.fori_loop(0, n_super, super_round, 0)

        plsc.subcore_barrier()

        with scope("sc_writeout"):
            pltpu.sync_copy(
                acc.at[pl.ds(row0, rows_pt)],
                out_hbm.at[pl.ds(
                    pl.multiple_of(cid * n_nodes + row0, 8), rows_pt)])
            if tail:
                @pl.when(sid == NS - 1)
                def _():
                    pltpu.sync_copy(
                        acc.at[pl.ds(NS * rows_pt, tail)],
                        out_hbm.at[pl.ds(pl.multiple_of(
                            cid * n_nodes + NS * rows_pt, 8), tail)])

    return k


def _tc_add(n_nodes, d, blk):
    def body(a_ref, b_ref, o_ref):
        o_ref[...] = a_ref[...] + b_ref[...]

    return pl.pallas_call(
        body,
        grid=(n_nodes // blk,),
        in_specs=[pl.BlockSpec((blk, d), lambda i: (i, 0))] * 2,
        out_specs=pl.BlockSpec((blk, d), lambda i: (i, 0)),
        out_shape=jax.ShapeDtypeStruct((n_nodes, d), jnp.float32),
    )


@jax.jit
def kernel(input, sidx, tidx, enorm, esgn):
    n_nodes, d = input.shape
    n_edges = sidx.shape[0]
    c_sz = 80
    sc_chunks = 25

    n_super = n_edges // NW // c_sz // sc_chunks
    sidx3 = sidx.astype(jnp.int32).reshape(NW, n_super, sc_chunks, c_sz)
    tidx3 = tidx.astype(jnp.int32).reshape(NW, n_super, sc_chunks, c_sz)

    partials = _sc_scatter_gather(n_nodes, n_edges, d, c_sz, sc_chunks)(
        input, sidx3, tidx3, enorm, esgn)
    return _tc_add(n_nodes, d, 1000)(partials[:n_nodes], partials[n_nodes:])


# speedup vs baseline: 1.4426x; 1.4426x over previous
import functools

import jax
import jax.numpy as jnp
from jax import lax
from jax.experimental import pallas as pl
from jax.experimental.pallas import tpu as pltpu
from jax.experimental.pallas import tpu_sc as plsc

NC = 2
NS = 16
NW = NC * NS
L = 16


def _sc_scatter_gather(n_nodes, n_edges, d, c_sz, sc_chunks):
    epw = n_edges // NW
    n_chunks = epw // c_sz
    n_super = n_chunks // sc_chunks
    s_sz = sc_chunks * c_sz
    rows_pt = (n_nodes // NS) // 8 * 8
    tail = n_nodes - rows_pt * NS
    zrows = 8
    mesh = plsc.VectorSubcoreMesh(core_axis_name="c", subcore_axis_name="s")

    @functools.partial(
        pl.kernel,
        out_type=jax.ShapeDtypeStruct((NC * n_nodes, d), jnp.float32),
        mesh=mesh,
        scratch_types=[
            pltpu.VMEM((sc_chunks, c_sz), jnp.int32),
            pltpu.VMEM((sc_chunks, c_sz), jnp.int32),
            pltpu.VMEM((s_sz,), jnp.float32),
            pltpu.VMEM((s_sz,), jnp.float32),
            pltpu.VMEM((c_sz, d), jnp.float32),
            pltpu.VMEM((c_sz, d), jnp.float32),
            pltpu.VMEM((c_sz, d), jnp.float32),
            pltpu.VMEM((zrows, d), jnp.float32),
            pltpu.VMEM_SHARED((n_nodes, d), jnp.float32),
            pltpu.SemaphoreType.DMA,
            pltpu.SemaphoreType.DMA,
            pltpu.SemaphoreType.DMA,
            pltpu.SemaphoreType.DMA,
            pltpu.SemaphoreType.DMA,
            pltpu.SemaphoreType.DMA,
        ],
    )
    def k(inp_hbm, sidx_hbm, tidx_hbm, enorm_hbm, esgn_hbm, out_hbm,
          sidx_v, tidx_v, w_v, sg_v, rows_0, rows_1, rows_2, zbuf, acc,
          gs_0, gs_1, gs_2, ss_0, ss_1, ss_2):
        rows = (rows_0, rows_1, rows_2)
        gs = (gs_0, gs_1, gs_2)
        ss = (ss_0, ss_1, ss_2)
        cid = lax.axis_index("c")
        sid = lax.axis_index("s")
        wid = sid * NC + cid
        row0 = pl.multiple_of(sid * rows_pt, 8)
        scope = jax.named_scope

        sc_init = scope("sc_init")
        sc_init.__enter__()
        def zero_zbuf(r, _):
            for f in range(d // L):
                zbuf[r, pl.ds(f * L, L)] = jnp.zeros((L,), jnp.float32)
            return 0
        lax.fori_loop(0, zrows, zero_zbuf, 0)

        def zero_acc(i, _):
            pltpu.sync_copy(
                zbuf, acc.at[pl.ds(pl.multiple_of(row0 + i * zrows, 8), zrows)])
            return 0
        lax.fori_loop(0, rows_pt // zrows, zero_acc, 0)
        if tail:
            @pl.when(sid == NS - 1)
            def _():
                for tpart in range(0, tail, zrows):
                    pltpu.sync_copy(
                        zbuf, acc.at[pl.ds(NS * rows_pt + tpart, zrows)])

        plsc.subcore_barrier()
        sc_init.__exit__(None, None, None)

        def scale(rows_v, c):
            def escale(g, _):
                w16 = w_v[pl.ds(c * c_sz + g * L, L)]
                for j in range(L):
                    wb = lax.gather(
                        w16, jnp.full((L, 1), j, jnp.int32),
                        lax.GatherDimensionNumbers(
                            offset_dims=(), collapsed_slice_dims=(0,),
                            start_index_map=(0,)),
                        (1,), mode=lax.GatherScatterMode.PROMISE_IN_BOUNDS)
                    e = g * L + j
                    for f in range(d // L):
                        rows_v[e, pl.ds(f * L, L)] = (
                            rows_v[e, pl.ds(f * L, L)] * wb)
                return 0
            lax.fori_loop(0, c_sz // L, escale, 0)

        def issue_gather(c, rows_v, sem):
            pltpu.async_copy(inp_hbm.at[sidx_v.at[c]], rows_v, sem)

        def wait_gather(rows_v, sem):
            pltpu.make_async_copy(inp_hbm.at[sidx_v.at[0]], rows_v, sem).wait()

        def issue_scatter(c, rows_v, sem):
            pltpu.async_copy(rows_v, acc.at[tidx_v.at[c]], sem, add=True)

        def wait_scatter(rows_v, sem):
            pltpu.make_async_copy(rows_v, acc.at[tidx_v.at[0]], sem).wait()

        def super_round(s, _):
            with scope("sc_stage"):
                pltpu.sync_copy(sidx_hbm.at[wid, s], sidx_v)
                pltpu.sync_copy(tidx_hbm.at[wid, s], tidx_v)
                e0 = pl.multiple_of(wid * epw + s * s_sz, 8)
                pltpu.sync_copy(enorm_hbm.at[pl.ds(e0, s_sz)], w_v)
                pltpu.sync_copy(esgn_hbm.at[pl.ds(e0, s_sz)], sg_v)

                def wmul(kk, _):
                    w_v[pl.ds(kk * L, L)] = (
                        w_v[pl.ds(kk * L, L)] * sg_v[pl.ds(kk * L, L)])
                    return 0
                lax.fori_loop(0, s_sz // L, wmul, 0)

            issue_gather(0, rows[0], gs[0])
            if sc_chunks > 1:
                issue_gather(1, rows[1], gs[1])

            def phase(c, b):
                @pl.when(c < sc_chunks)
                def _():
                    wait_gather(rows[b], gs[b])
                    scale(rows[b], c)
                    b2 = (b + 2) % 3
                    @pl.when(c + 2 < sc_chunks)
                    def _():
                        @pl.when(c >= 1)
                        def _():
                            wait_scatter(rows[b2], ss[b2])
                        issue_gather(c + 2, rows[b2], gs[b2])
                    issue_scatter(c, rows[b], ss[b])

            def triple(t, _):
                c0 = 3 * t
                phase(c0, 0)
                phase(c0 + 1, 1)
                phase(c0 + 2, 2)
                return 0
            lax.fori_loop(0, (sc_chunks + 2) // 3, triple, 0)

            for b in range(min(3, sc_chunks)):
                wait_scatter(rows[b], ss[b])
            return 0
        lax.fori_loop(0, n_super, super_round, 0)

        plsc.subcore_barrier()

        with scope("sc_writeout"):
            pltpu.sync_copy(
                acc.at[pl.ds(row0, rows_pt)],
                out_hbm.at[pl.ds(
                    pl.multiple_of(cid * n_nodes + row0, 8), rows_pt)])
            if tail:
                @pl.when(sid == NS - 1)
                def _():
                    pltpu.sync_copy(
                        acc.at[pl.ds(NS * rows_pt, tail)],
                        out_hbm.at[pl.ds(pl.multiple_of(
                            cid * n_nodes + NS * rows_pt, 8), tail)])

    return k


def _tc_add(n_nodes, d, blk):
    def body(a_ref, b_ref, o_ref):
        o_ref[...] = a_ref[...] + b_ref[...]

    return pl.pallas_call(
        body,
        grid=(n_nodes // blk,),
        in_specs=[pl.BlockSpec((blk, d), lambda i: (i, 0))] * 2,
        out_specs=pl.BlockSpec((blk, d), lambda i: (i, 0)),
        out_shape=jax.ShapeDtypeStruct((n_nodes, d), jnp.float32),
    )


@jax.jit
def kernel(input, sidx, tidx, enorm, esgn):
    n_nodes, d = input.shape
    n_edges = sidx.shape[0]
    c_sz = 80
    sc_chunks = 25

    n_super = n_edges // NW // c_sz // sc_chunks
    sidx3 = sidx.astype(jnp.int32).reshape(NW, n_super, sc_chunks, c_sz)
    tidx3 = tidx.astype(jnp.int32).reshape(NW, n_super, sc_chunks, c_sz)

    partials = _sc_scatter_gather(n_nodes, n_edges, d, c_sz, sc_chunks)(
        input, sidx3, tidx3, enorm, esgn)
    return _tc_add(n_nodes, d, 1000)(partials[:n_nodes], partials[n_nodes:])
